# Initial kernel scaffold; baseline (speedup 1.0000x reference)
import jax
import jax.numpy as jnp
from jax.experimental import pallas as pl


def kernel(x, edge_index, edge_weight, W1, W2):
    raise NotImplementedError("write your pallas kernel here")


# speedup vs baseline: 5.6694x; 5.6694x over previous
import functools

import jax
import jax.numpy as jnp
from jax import lax
from jax.experimental import pallas as pl
from jax.experimental.pallas import tpu as pltpu
from jax.experimental.pallas import tpu_sc as plsc

N_NODES = 10000
N_EDGES = 320000
D_FEAT = 128
HIDDEN1 = 16
OUT_DIM = 64

B = 128
NBLK = N_EDGES // B
NC = 2
NS = 16
NW = NC * NS
ROWS_PER_TILE = 632
N_PAD = NS * ROWS_PER_TILE
LANES = 16


def _sc_seg_body(F, h_hbm, src_hbm, dst_hbm, ew_hbm, zeros_hbm, out_hbm,
                 acc, src_v, dst_v, ew_v, rows_v, sem):
    cid = lax.axis_index("c")
    sid = lax.axis_index("s")
    wid = sid * NC + cid

    r0 = sid * ROWS_PER_TILE
    pltpu.sync_copy(zeros_hbm.at[pl.ds(r0, ROWS_PER_TILE)],
                    acc.at[pl.ds(r0, ROWS_PER_TILE)])
    plsc.subcore_barrier()

    lo = (wid * NBLK) // NW
    hi = ((wid + 1) * NBLK) // NW

    def blk(g, carry):
        e0 = g * B
        pltpu.sync_copy(src_hbm.at[pl.ds(e0, B)], src_v)
        pltpu.sync_copy(dst_hbm.at[pl.ds(e0, B)], dst_v)
        pltpu.sync_copy(ew_hbm.at[pl.ds(e0, B)], ew_v)
        pltpu.async_copy(h_hbm.at[src_v], rows_v, sem).wait()

        def scale(i, c):
            w = plsc.load_gather(ew_v, [jnp.full((LANES,), i, jnp.int32)])
            for j in range(F // LANES):
                rows_v[i, pl.ds(j * LANES, LANES)] = (
                    rows_v[i, pl.ds(j * LANES, LANES)] * w)
            return c
        lax.fori_loop(0, B, scale, 0, unroll=4)

        pltpu.sync_copy(rows_v, acc.at[dst_v], add=True)
        return carry

    lax.fori_loop(lo, hi, blk, 0)

    plsc.subcore_barrier()
    pltpu.sync_copy(acc.at[pl.ds(r0, ROWS_PER_TILE)],
                    out_hbm.at[cid].at[pl.ds(r0, ROWS_PER_TILE)])


def _make_seg(F):
    return pl.kernel(
        functools.partial(_sc_seg_body, F),
        out_type=jax.ShapeDtypeStruct((NC, N_PAD, F), jnp.float32),
        mesh=plsc.VectorSubcoreMesh(core_axis_name="c", subcore_axis_name="s",
                                    num_cores=NC, num_subcores=NS),
        scratch_types=[
            pltpu.VMEM_SHARED((N_PAD, F), jnp.float32),
            pltpu.VMEM((B,), jnp.int32),
            pltpu.VMEM((B,), jnp.int32),
            pltpu.VMEM((B,), jnp.float32),
            pltpu.VMEM((B, F), jnp.float32),
            pltpu.SemaphoreType.DMA,
        ],
        compiler_params=pltpu.CompilerParams(needs_layout_passes=False,
                                             use_tc_tiling_on_sc=False),
    )


_seg16 = _make_seg(HIDDEN1)
_seg64 = _make_seg(OUT_DIM)


def _mm1_body(x_ref, w_ref, o_ref):
    o_ref[...] = jnp.dot(x_ref[...], w_ref[...],
                         preferred_element_type=jnp.float32)


_mm1 = pl.pallas_call(
    _mm1_body,
    out_shape=jax.ShapeDtypeStruct((N_NODES, HIDDEN1), jnp.float32),
)


def _mid_body(p_ref, w_ref, o_ref):
    h1 = jnp.maximum(p_ref[0] + p_ref[1], 0.0)
    o_ref[...] = jnp.dot(h1, w_ref[...], preferred_element_type=jnp.float32)


_mid = pl.pallas_call(
    _mid_body,
    out_shape=jax.ShapeDtypeStruct((N_NODES, OUT_DIM), jnp.float32),
)


def _soft_body(p_ref, o_ref):
    h = p_ref[0] + p_ref[1]
    m = jnp.max(h, axis=1, keepdims=True)
    e = jnp.exp(h - m)
    o_ref[...] = e / jnp.sum(e, axis=1, keepdims=True)


_soft = pl.pallas_call(
    _soft_body,
    out_shape=jax.ShapeDtypeStruct((N_NODES, OUT_DIM), jnp.float32),
)


@jax.jit
def _impl(x, src, dst, ew, W1, W2):
    zeros16 = jnp.zeros((N_PAD, HIDDEN1), jnp.float32)
    zeros64 = jnp.zeros((N_PAD, OUT_DIM), jnp.float32)
    hw1 = _mm1(x, W1)
    p1 = _seg16(hw1, src, dst, ew, zeros16)
    hw2 = _mid(p1[:, :N_NODES], W2)
    p2 = _seg64(hw2, src, dst, ew, zeros64)
    return _soft(p2[:, :N_NODES])


def kernel(x, edge_index, edge_weight, W1, W2):
    src = edge_index[0].astype(jnp.int32)
    dst = edge_index[1].astype(jnp.int32)
    return _impl(x, src, dst, edge_weight, W1, W2)


# speedup vs baseline: 16.5749x; 2.9236x over previous
import functools

import jax
import jax.numpy as jnp
from jax import lax
from jax.experimental import pallas as pl
from jax.experimental.pallas import tpu as pltpu
from jax.experimental.pallas import tpu_sc as plsc

N_NODES = 10000
N_EDGES = 320000
D_FEAT = 128
HIDDEN1 = 16
OUT_DIM = 64

LANES = 16
CHUNK = 128
NCH = 4
SB = CHUNK * NCH
NSB = N_EDGES // SB
NC = 2
NS = 16
NW = NC * NS
ROWS_PER_TILE = 632
N_PAD = NS * ROWS_PER_TILE


def _sc_seg_body(F, h_hbm, src_hbm, dst_hbm, ew_hbm, zeros_hbm, out_hbm,
                 acc, src_v0, src_v1, dst_v0, dst_v1, ew_v0, ew_v1,
                 rows0, rows1, sem_i0, sem_i1, sem_g0, sem_g1):
    cid = lax.axis_index("c")
    sid = lax.axis_index("s")
    wid = sid * NC + cid

    src_v = (src_v0, src_v1)
    dst_v = (dst_v0, dst_v1)
    ew_v = (ew_v0, ew_v1)
    rows = (rows0, rows1)
    sem_i = (sem_i0, sem_i1)
    sem_g = (sem_g0, sem_g1)

    r0 = sid * ROWS_PER_TILE
    pltpu.sync_copy(zeros_hbm.at[pl.ds(r0, ROWS_PER_TILE)],
                    acc.at[pl.ds(r0, ROWS_PER_TILE)])
    plsc.subcore_barrier()

    lo = (wid * NSB) // NW
    hi = ((wid + 1) * NSB) // NW

    def fire_idx(k, b):
        pltpu.async_copy(src_hbm.at[pl.ds(k * NCH, NCH)], src_v[b], sem_i[b])
        pltpu.async_copy(dst_hbm.at[pl.ds(k * NCH, NCH)], dst_v[b], sem_i[b])
        pltpu.async_copy(ew_hbm.at[pl.ds(k * SB, SB)], ew_v[b], sem_i[b])

    def wait_idx(b):
        pltpu.make_async_copy(src_hbm.at[pl.ds(0, NCH)], src_v[b],
                              sem_i[b]).wait()
        pltpu.make_async_copy(dst_hbm.at[pl.ds(0, NCH)], dst_v[b],
                              sem_i[b]).wait()
        pltpu.make_async_copy(ew_hbm.at[pl.ds(0, SB)], ew_v[b],
                              sem_i[b]).wait()

    def fire_gather(b):
        for j in range(NCH):
            pltpu.async_copy(h_hbm.at[src_v[b].at[j]],
                             rows[b].at[pl.ds(j * CHUNK, CHUNK)], sem_g[b])

    def wait_gather(b):
        for j in range(NCH):
            pltpu.make_async_copy(h_hbm.at[src_v[b].at[j]],
                                  rows[b].at[pl.ds(j * CHUNK, CHUNK)],
                                  sem_g[b]).wait()

    def half(k, b):
        wait_gather(b)

        @pl.when(k + 1 < hi)
        def _():
            wait_idx(1 - b)
            fire_gather(1 - b)

        r = rows[b]
        w_ref = ew_v[b]

        @plsc.parallel_loop(0, SB, 1, unroll=8)
        def _(i):
            w = plsc.load_gather(w_ref, [jnp.full((LANES,), i, jnp.int32)])
            for j in range(F // LANES):
                r[i, pl.ds(j * LANES, LANES)] = (
                    r[i, pl.ds(j * LANES, LANES)] * w)

        for j in range(NCH):
            pltpu.sync_copy(rows[b].at[pl.ds(j * CHUNK, CHUNK)],
                            acc.at[dst_v[b].at[j]], add=True)

        @pl.when(k + 2 < hi)
        def _():
            fire_idx(k + 2, b)

    fire_idx(lo, 0)
    wait_idx(0)
    fire_gather(0)
    fire_idx(lo + 1, 1)

    n_t = (hi - lo + 1) // 2

    def t_body(t, carry):
        k0 = lo + 2 * t
        half(k0, 0)

        @pl.when(k0 + 1 < hi)
        def _():
            half(k0 + 1, 1)
        return carry

    lax.fori_loop(0, n_t, t_body, 0)

    plsc.subcore_barrier()
    pltpu.sync_copy(acc.at[pl.ds(r0, ROWS_PER_TILE)],
                    out_hbm.at[cid].at[pl.ds(r0, ROWS_PER_TILE)])


def _make_seg(F):
    return pl.kernel(
        functools.partial(_sc_seg_body, F),
        out_type=jax.ShapeDtypeStruct((NC, N_PAD, F), jnp.float32),
        mesh=plsc.VectorSubcoreMesh(core_axis_name="c", subcore_axis_name="s",
                                    num_cores=NC, num_subcores=NS),
        scratch_types=[
            pltpu.VMEM_SHARED((N_PAD, F), jnp.float32),
            pltpu.VMEM((NCH, CHUNK), jnp.int32),
            pltpu.VMEM((NCH, CHUNK), jnp.int32),
            pltpu.VMEM((NCH, CHUNK), jnp.int32),
            pltpu.VMEM((NCH, CHUNK), jnp.int32),
            pltpu.VMEM((SB,), jnp.float32),
            pltpu.VMEM((SB,), jnp.float32),
            pltpu.VMEM((SB, F), jnp.float32),
            pltpu.VMEM((SB, F), jnp.float32),
            pltpu.SemaphoreType.DMA,
            pltpu.SemaphoreType.DMA,
            pltpu.SemaphoreType.DMA,
            pltpu.SemaphoreType.DMA,
        ],
        compiler_params=pltpu.CompilerParams(needs_layout_passes=False,
                                             use_tc_tiling_on_sc=False),
    )


_seg16 = _make_seg(HIDDEN1)
_seg64 = _make_seg(OUT_DIM)


def _mm1_body(x_ref, w_ref, o_ref):
    o_ref[...] = jnp.dot(x_ref[...], w_ref[...],
                         preferred_element_type=jnp.float32)


_mm1 = pl.pallas_call(
    _mm1_body,
    out_shape=jax.ShapeDtypeStruct((N_NODES, HIDDEN1), jnp.float32),
)


def _mid_body(p_ref, w_ref, o_ref):
    h1 = jnp.maximum(p_ref[0, :N_NODES] + p_ref[1, :N_NODES], 0.0)
    o_ref[...] = jnp.dot(h1, w_ref[...], preferred_element_type=jnp.float32)


_mid = pl.pallas_call(
    _mid_body,
    out_shape=jax.ShapeDtypeStruct((N_NODES, OUT_DIM), jnp.float32),
)


def _soft_body(p_ref, o_ref):
    h = p_ref[0, :N_NODES] + p_ref[1, :N_NODES]
    m = jnp.max(h, axis=1, keepdims=True)
    e = jnp.exp(h - m)
    o_ref[...] = e / jnp.sum(e, axis=1, keepdims=True)


_soft = pl.pallas_call(
    _soft_body,
    out_shape=jax.ShapeDtypeStruct((N_NODES, OUT_DIM), jnp.float32),
)


@jax.jit
def _impl(x, src, dst, ew, W1, W2):
    src2 = src.reshape(NSB * NCH, CHUNK)
    dst2 = dst.reshape(NSB * NCH, CHUNK)
    zeros16 = jnp.zeros((N_PAD, HIDDEN1), jnp.float32)
    zeros64 = jnp.zeros((N_PAD, OUT_DIM), jnp.float32)
    hw1 = _mm1(x, W1)
    p1 = _seg16(hw1, src2, dst2, ew, zeros16)
    hw2 = _mid(p1, W2)
    p2 = _seg64(hw2, src2, dst2, ew, zeros64)
    return _soft(p2)


def kernel(x, edge_index, edge_weight, W1, W2):
    src = edge_index[0].astype(jnp.int32)
    dst = edge_index[1].astype(jnp.int32)
    return _impl(x, src, dst, edge_weight, W1, W2)


# speedup vs baseline: 17.3645x; 1.0476x over previous
import functools

import jax
import jax.numpy as jnp
from jax import lax
from jax.experimental import pallas as pl
from jax.experimental.pallas import tpu as pltpu
from jax.experimental.pallas import tpu_sc as plsc

N_NODES = 10000
N_EDGES = 320000
D_FEAT = 128
HIDDEN1 = 16
OUT_DIM = 64

LANES = 16
NC = 2
NS = 16
NW = NC * NS
ROWS_PER_TILE = 632
N_PAD = NS * ROWS_PER_TILE


def _sc_seg_body(F, CHUNK, NCH, h_hbm, src_hbm, dst_hbm, ew_hbm, zeros_hbm,
                 out_hbm, acc, src_v0, src_v1, dst_v0, dst_v1, ew_v0, ew_v1,
                 rows0, rows1, sem_i0, sem_i1, sem_g0, sem_g1, sem_s0,
                 sem_s1):
    SB = CHUNK * NCH
    NSB = N_EDGES // SB
    cid = lax.axis_index("c")
    sid = lax.axis_index("s")
    wid = sid * NC + cid

    src_v = (src_v0, src_v1)
    dst_v = (dst_v0, dst_v1)
    ew_v = (ew_v0, ew_v1)
    rows = (rows0, rows1)
    sem_i = (sem_i0, sem_i1)
    sem_g = (sem_g0, sem_g1)
    sem_s = (sem_s0, sem_s1)

    r0 = sid * ROWS_PER_TILE
    lo = (wid * NSB) // NW
    hi = ((wid + 1) * NSB) // NW

    def fire_idx(k, b):
        pltpu.async_copy(src_hbm.at[pl.ds(k * NCH, NCH)], src_v[b], sem_i[b])
        pltpu.async_copy(dst_hbm.at[pl.ds(k * NCH, NCH)], dst_v[b], sem_i[b])
        pltpu.async_copy(ew_hbm.at[pl.ds(k * SB, SB)], ew_v[b], sem_i[b])

    def wait_idx(b):
        pltpu.make_async_copy(src_hbm.at[pl.ds(0, NCH)], src_v[b],
                              sem_i[b]).wait()
        pltpu.make_async_copy(dst_hbm.at[pl.ds(0, NCH)], dst_v[b],
                              sem_i[b]).wait()
        pltpu.make_async_copy(ew_hbm.at[pl.ds(0, SB)], ew_v[b],
                              sem_i[b]).wait()

    def fire_gather(b):
        for j in range(NCH):
            pltpu.async_copy(h_hbm.at[src_v[b].at[j]],
                             rows[b].at[pl.ds(j * CHUNK, CHUNK)], sem_g[b])

    def wait_gather(b):
        for j in range(NCH):
            pltpu.make_async_copy(h_hbm.at[src_v[b].at[j]],
                                  rows[b].at[pl.ds(j * CHUNK, CHUNK)],
                                  sem_g[b]).wait()

    def half(k, b):
        wait_gather(b)

        @pl.when(k + 1 < hi)
        def _():
            wait_idx(1 - b)
            fire_gather(1 - b)

        r = rows[b]
        w_ref = ew_v[b]

        @plsc.parallel_loop(0, SB, 1, unroll=8)
        def _(i):
            w = plsc.load_gather(w_ref, [jnp.full((LANES,), i, jnp.int32)])
            for j in range(F // LANES):
                r[i, pl.ds(j * LANES, LANES)] = (
                    r[i, pl.ds(j * LANES, LANES)] * w)

        for j in range(NCH):
            pltpu.async_copy(rows[b].at[pl.ds(j * CHUNK, CHUNK)],
                             acc.at[dst_v[b].at[j]], sem_s[b], add=True)
        for j in range(NCH):
            pltpu.make_async_copy(rows[b].at[pl.ds(j * CHUNK, CHUNK)],
                                  acc.at[dst_v[b].at[j]], sem_s[b]).wait()

        @pl.when(k + 2 < hi)
        def _():
            fire_idx(k + 2, b)

    fire_idx(lo, 0)
    pltpu.sync_copy(zeros_hbm.at[pl.ds(r0, ROWS_PER_TILE)],
                    acc.at[pl.ds(r0, ROWS_PER_TILE)])
    wait_idx(0)
    fire_gather(0)
    fire_idx(lo + 1, 1)
    plsc.subcore_barrier()

    n_t = (hi - lo + 1) // 2

    def t_body(t, carry):
        k0 = lo + 2 * t
        half(k0, 0)

        @pl.when(k0 + 1 < hi)
        def _():
            half(k0 + 1, 1)
        return carry

    lax.fori_loop(0, n_t, t_body, 0)

    plsc.subcore_barrier()
    pltpu.sync_copy(acc.at[pl.ds(r0, ROWS_PER_TILE)],
                    out_hbm.at[cid].at[pl.ds(r0, ROWS_PER_TILE)])


def _make_seg(F, CHUNK, NCH):
    SB = CHUNK * NCH
    return pl.kernel(
        functools.partial(_sc_seg_body, F, CHUNK, NCH),
        out_type=jax.ShapeDtypeStruct((NC, N_PAD, F), jnp.float32),
        mesh=plsc.VectorSubcoreMesh(core_axis_name="c", subcore_axis_name="s",
                                    num_cores=NC, num_subcores=NS),
        scratch_types=[
            pltpu.VMEM_SHARED((N_PAD, F), jnp.float32),
            pltpu.VMEM((NCH, CHUNK), jnp.int32),
            pltpu.VMEM((NCH, CHUNK), jnp.int32),
            pltpu.VMEM((NCH, CHUNK), jnp.int32),
            pltpu.VMEM((NCH, CHUNK), jnp.int32),
            pltpu.VMEM((SB,), jnp.float32),
            pltpu.VMEM((SB,), jnp.float32),
            pltpu.VMEM((SB, F), jnp.float32),
            pltpu.VMEM((SB, F), jnp.float32),
            pltpu.SemaphoreType.DMA,
            pltpu.SemaphoreType.DMA,
            pltpu.SemaphoreType.DMA,
            pltpu.SemaphoreType.DMA,
            pltpu.SemaphoreType.DMA,
            pltpu.SemaphoreType.DMA,
        ],
        compiler_params=pltpu.CompilerParams(needs_layout_passes=False,
                                             use_tc_tiling_on_sc=False),
    )


CHUNK16, NCH16 = 125, 16
CHUNK64, NCH64 = 128, 4
_seg16 = _make_seg(HIDDEN1, CHUNK16, NCH16)
_seg64 = _make_seg(OUT_DIM, CHUNK64, NCH64)


def _mm1_body(x_ref, w_ref, o_ref):
    o_ref[...] = jnp.dot(x_ref[...], w_ref[...],
                         preferred_element_type=jnp.float32)


_mm1 = pl.pallas_call(
    _mm1_body,
    out_shape=jax.ShapeDtypeStruct((N_NODES, HIDDEN1), jnp.float32),
)


def _mid_body(p_ref, w_ref, o_ref):
    h1 = jnp.maximum(p_ref[0, :N_NODES] + p_ref[1, :N_NODES], 0.0)
    o_ref[...] = jnp.dot(h1, w_ref[...], preferred_element_type=jnp.float32)


_mid = pl.pallas_call(
    _mid_body,
    out_shape=jax.ShapeDtypeStruct((N_NODES, OUT_DIM), jnp.float32),
)


def _soft_body(p_ref, o_ref):
    h = p_ref[0, :N_NODES] + p_ref[1, :N_NODES]
    m = jnp.max(h, axis=1, keepdims=True)
    e = jnp.exp(h - m)
    o_ref[...] = e / jnp.sum(e, axis=1, keepdims=True)


_soft = pl.pallas_call(
    _soft_body,
    out_shape=jax.ShapeDtypeStruct((N_NODES, OUT_DIM), jnp.float32),
)


@jax.jit
def _impl(x, src, dst, ew, W1, W2):
    src16 = src.reshape(N_EDGES // CHUNK16, CHUNK16)
    dst16 = dst.reshape(N_EDGES // CHUNK16, CHUNK16)
    src64 = src.reshape(N_EDGES // CHUNK64, CHUNK64)
    dst64 = dst.reshape(N_EDGES // CHUNK64, CHUNK64)
    zeros16 = jnp.zeros((N_PAD, HIDDEN1), jnp.float32)
    zeros64 = jnp.zeros((N_PAD, OUT_DIM), jnp.float32)
    hw1 = _mm1(x, W1)
    p1 = _seg16(hw1, src16, dst16, ew, zeros16)
    hw2 = _mid(p1, W2)
    p2 = _seg64(hw2, src64, dst64, ew, zeros64)
    return _soft(p2)


def kernel(x, edge_index, edge_weight, W1, W2):
    src = edge_index[0].astype(jnp.int32)
    dst = edge_index[1].astype(jnp.int32)
    return _impl(x, src, dst, edge_weight, W1, W2)


# speedup vs baseline: 17.4639x; 1.0057x over previous
import functools

import jax
import jax.numpy as jnp
from jax import lax
from jax.experimental import pallas as pl
from jax.experimental.pallas import tpu as pltpu
from jax.experimental.pallas import tpu_sc as plsc

N_NODES = 10000
N_EDGES = 320000
D_FEAT = 128
HIDDEN1 = 16
OUT_DIM = 64

LANES = 16
NC = 2
NS = 16
NW = NC * NS
ROWS_PER_TILE = 632
N_PAD = NS * ROWS_PER_TILE


def _sc_seg_body(F, CHUNK, NCH, h_hbm, src_hbm, dst_hbm, ew_hbm,
                 out_hbm, acc, src_v0, src_v1, dst_v0, dst_v1, ew_v0,
                 ew_v1, rows0, rows1, sem_i0, sem_i1, sem_g0, sem_g1, sem_s0,
                 sem_s1):
    SB = CHUNK * NCH
    NSB = N_EDGES // SB
    cid = lax.axis_index("c")
    sid = lax.axis_index("s")
    wid = sid * NC + cid

    src_v = (src_v0, src_v1)
    dst_v = (dst_v0, dst_v1)
    ew_v = (ew_v0, ew_v1)
    rows = (rows0, rows1)
    sem_i = (sem_i0, sem_i1)
    sem_g = (sem_g0, sem_g1)
    sem_s = (sem_s0, sem_s1)

    r0 = sid * ROWS_PER_TILE
    lo = (wid * NSB) // NW
    hi = ((wid + 1) * NSB) // NW

    def fire_idx(k, b):
        pltpu.async_copy(src_hbm.at[pl.ds(k * NCH, NCH)], src_v[b], sem_i[b])
        pltpu.async_copy(dst_hbm.at[pl.ds(k * NCH, NCH)], dst_v[b], sem_i[b])
        pltpu.async_copy(ew_hbm.at[pl.ds(k * SB, SB)], ew_v[b], sem_i[b])

    def wait_idx(b):
        pltpu.make_async_copy(src_hbm.at[pl.ds(0, NCH)], src_v[b],
                              sem_i[b]).wait()
        pltpu.make_async_copy(dst_hbm.at[pl.ds(0, NCH)], dst_v[b],
                              sem_i[b]).wait()
        pltpu.make_async_copy(ew_hbm.at[pl.ds(0, SB)], ew_v[b],
                              sem_i[b]).wait()

    def fire_gather(b):
        for j in range(NCH):
            pltpu.async_copy(h_hbm.at[src_v[b].at[j]],
                             rows[b].at[pl.ds(j * CHUNK, CHUNK)], sem_g[b])

    def wait_gather(b):
        for j in range(NCH):
            pltpu.make_async_copy(h_hbm.at[src_v[b].at[j]],
                                  rows[b].at[pl.ds(j * CHUNK, CHUNK)],
                                  sem_g[b]).wait()

    def half(k, b):
        wait_gather(b)

        @pl.when(k + 1 < hi)
        def _():
            wait_idx(1 - b)
            fire_gather(1 - b)

        r = rows[b]
        w_ref = ew_v[b]

        @plsc.parallel_loop(0, SB, 1, unroll=8)
        def _(i):
            w = plsc.load_gather(w_ref, [jnp.full((LANES,), i, jnp.int32)])
            for j in range(F // LANES):
                r[i, pl.ds(j * LANES, LANES)] = (
                    r[i, pl.ds(j * LANES, LANES)] * w)

        for j in range(NCH):
            pltpu.async_copy(rows[b].at[pl.ds(j * CHUNK, CHUNK)],
                             acc.at[dst_v[b].at[j]], sem_s[b], add=True)
        for j in range(NCH):
            pltpu.make_async_copy(rows[b].at[pl.ds(j * CHUNK, CHUNK)],
                                  acc.at[dst_v[b].at[j]], sem_s[b]).wait()

        @pl.when(k + 2 < hi)
        def _():
            fire_idx(k + 2, b)

    fire_idx(lo, 0)
    zvec = jnp.zeros((LANES,), jnp.float32)
    z1 = min(SB, ROWS_PER_TILE)
    z2 = ROWS_PER_TILE - z1

    @plsc.parallel_loop(0, z1, 1, unroll=8)
    def _(i):
        for j in range(F // LANES):
            rows0[i, pl.ds(j * LANES, LANES)] = zvec

    if z2:
        @plsc.parallel_loop(0, z2, 1, unroll=8)
        def _(i):
            for j in range(F // LANES):
                rows1[i, pl.ds(j * LANES, LANES)] = zvec

    pltpu.sync_copy(rows0.at[pl.ds(0, z1)], acc.at[pl.ds(r0, z1)])
    if z2:
        pltpu.sync_copy(rows1.at[pl.ds(0, z2)], acc.at[pl.ds(r0 + z1, z2)])
    wait_idx(0)
    fire_gather(0)
    fire_idx(lo + 1, 1)
    plsc.subcore_barrier()

    n_t = (hi - lo + 1) // 2

    def t_body(t, carry):
        k0 = lo + 2 * t
        half(k0, 0)

        @pl.when(k0 + 1 < hi)
        def _():
            half(k0 + 1, 1)
        return carry

    lax.fori_loop(0, n_t, t_body, 0)

    plsc.subcore_barrier()
    pltpu.sync_copy(acc.at[pl.ds(r0, ROWS_PER_TILE)],
                    out_hbm.at[cid].at[pl.ds(r0, ROWS_PER_TILE)])


def _make_seg(F, CHUNK, NCH):
    SB = CHUNK * NCH
    return pl.kernel(
        functools.partial(_sc_seg_body, F, CHUNK, NCH),
        out_type=jax.ShapeDtypeStruct((NC, N_PAD, F), jnp.float32),
        mesh=plsc.VectorSubcoreMesh(core_axis_name="c", subcore_axis_name="s",
                                    num_cores=NC, num_subcores=NS),
        scratch_types=[
            pltpu.VMEM_SHARED((N_PAD, F), jnp.float32),
            pltpu.VMEM((NCH, CHUNK), jnp.int32),
            pltpu.VMEM((NCH, CHUNK), jnp.int32),
            pltpu.VMEM((NCH, CHUNK), jnp.int32),
            pltpu.VMEM((NCH, CHUNK), jnp.int32),
            pltpu.VMEM((SB,), jnp.float32),
            pltpu.VMEM((SB,), jnp.float32),
            pltpu.VMEM((SB, F), jnp.float32),
            pltpu.VMEM((SB, F), jnp.float32),
            pltpu.SemaphoreType.DMA,
            pltpu.SemaphoreType.DMA,
            pltpu.SemaphoreType.DMA,
            pltpu.SemaphoreType.DMA,
            pltpu.SemaphoreType.DMA,
            pltpu.SemaphoreType.DMA,
        ],
        compiler_params=pltpu.CompilerParams(needs_layout_passes=False,
                                             use_tc_tiling_on_sc=False,
                                             skip_device_barrier=True),
    )


CHUNK16, NCH16 = 125, 16
CHUNK64, NCH64 = 128, 4
_seg16 = _make_seg(HIDDEN1, CHUNK16, NCH16)
_seg64 = _make_seg(OUT_DIM, CHUNK64, NCH64)


def _mm1_body(x_ref, w_ref, o_ref):
    o_ref[...] = jnp.dot(x_ref[...], w_ref[...],
                         preferred_element_type=jnp.float32)


_mm1 = pl.pallas_call(
    _mm1_body,
    out_shape=jax.ShapeDtypeStruct((N_NODES, HIDDEN1), jnp.float32),
)


def _mid_body(p_ref, w_ref, o_ref):
    h1 = jnp.maximum(p_ref[0, :N_NODES] + p_ref[1, :N_NODES], 0.0)
    o_ref[...] = jnp.dot(h1, w_ref[...], preferred_element_type=jnp.float32)


_mid = pl.pallas_call(
    _mid_body,
    out_shape=jax.ShapeDtypeStruct((N_NODES, OUT_DIM), jnp.float32),
)


def _soft_body(p_ref, o_ref):
    h = p_ref[0, :N_NODES] + p_ref[1, :N_NODES]
    m = jnp.max(h, axis=1, keepdims=True)
    e = jnp.exp(h - m)
    o_ref[...] = e / jnp.sum(e, axis=1, keepdims=True)


_soft = pl.pallas_call(
    _soft_body,
    out_shape=jax.ShapeDtypeStruct((N_NODES, OUT_DIM), jnp.float32),
)


@jax.jit
def _impl(x, src, dst, ew, W1, W2):
    src16 = src.reshape(N_EDGES // CHUNK16, CHUNK16)
    dst16 = dst.reshape(N_EDGES // CHUNK16, CHUNK16)
    src64 = src.reshape(N_EDGES // CHUNK64, CHUNK64)
    dst64 = dst.reshape(N_EDGES // CHUNK64, CHUNK64)
    hw1 = _mm1(x, W1)
    p1 = _seg16(hw1, src16, dst16, ew)
    hw2 = _mid(p1, W2)
    p2 = _seg64(hw2, src64, dst64, ew)
    return _soft(p2)


def kernel(x, edge_index, edge_weight, W1, W2):
    src = edge_index[0].astype(jnp.int32)
    dst = edge_index[1].astype(jnp.int32)
    return _impl(x, src, dst, edge_weight, W1, W2)


# speedup vs baseline: 18.9089x; 1.0827x over previous
import functools

import jax
import jax.numpy as jnp
from jax import lax
from jax.experimental import pallas as pl
from jax.experimental.pallas import tpu as pltpu
from jax.experimental.pallas import tpu_sc as plsc

N_NODES = 10000
N_EDGES = 320000
D_FEAT = 128
HIDDEN1 = 16
OUT_DIM = 64

LANES = 16
NC = 2
NS = 16
NW = NC * NS
ROWS_PER_TILE = 632
N_PAD = NS * ROWS_PER_TILE


def _sc_seg_body(F, CHUNK, NCH, h_hbm, src_hbm, dst_hbm, ew_hbm,
                 out_hbm, acc, src_v0, src_v1, src_v2, dst_v0, dst_v1, dst_v2,
                 ew_v0, ew_v1, ew_v2, rows0, rows1, rows2, sem_i0, sem_i1,
                 sem_i2, sem_g0, sem_g1, sem_g2, sem_s0, sem_s1, sem_s2):
    SB = CHUNK * NCH
    NSB = N_EDGES // SB
    cid = lax.axis_index("c")
    sid = lax.axis_index("s")
    wid = sid * NC + cid

    src_v = (src_v0, src_v1, src_v2)
    dst_v = (dst_v0, dst_v1, dst_v2)
    ew_v = (ew_v0, ew_v1, ew_v2)
    rows = (rows0, rows1, rows2)
    sem_i = (sem_i0, sem_i1, sem_i2)
    sem_g = (sem_g0, sem_g1, sem_g2)
    sem_s = (sem_s0, sem_s1, sem_s2)

    r0 = sid * ROWS_PER_TILE
    lo = (wid * NSB) // NW
    hi = ((wid + 1) * NSB) // NW

    def fire_idx(k, b):
        pltpu.async_copy(src_hbm.at[pl.ds(k * NCH, NCH)], src_v[b], sem_i[b])
        pltpu.async_copy(dst_hbm.at[pl.ds(k * NCH, NCH)], dst_v[b], sem_i[b])
        pltpu.async_copy(ew_hbm.at[pl.ds(k * SB, SB)], ew_v[b], sem_i[b])

    def wait_idx(b):
        pltpu.make_async_copy(src_hbm.at[pl.ds(0, NCH)], src_v[b],
                              sem_i[b]).wait()
        pltpu.make_async_copy(dst_hbm.at[pl.ds(0, NCH)], dst_v[b],
                              sem_i[b]).wait()
        pltpu.make_async_copy(ew_hbm.at[pl.ds(0, SB)], ew_v[b],
                              sem_i[b]).wait()

    def fire_gather(b):
        for j in range(NCH):
            pltpu.async_copy(h_hbm.at[src_v[b].at[j]],
                             rows[b].at[pl.ds(j * CHUNK, CHUNK)], sem_g[b])

    def wait_gather(b):
        for j in range(NCH):
            pltpu.make_async_copy(h_hbm.at[src_v[b].at[j]],
                                  rows[b].at[pl.ds(j * CHUNK, CHUNK)],
                                  sem_g[b]).wait()

    def fire_scatter(b):
        for j in range(NCH):
            pltpu.async_copy(rows[b].at[pl.ds(j * CHUNK, CHUNK)],
                             acc.at[dst_v[b].at[j]], sem_s[b], add=True)

    def drain_scatter(b):
        for j in range(NCH):
            pltpu.make_async_copy(rows[b].at[pl.ds(j * CHUNK, CHUNK)],
                                  acc.at[dst_v[b].at[j]], sem_s[b]).wait()

    def phase(k, b):
        wait_gather(b)

        @pl.when(k + 1 < hi)
        def _():
            wait_idx((b + 1) % 3)
            fire_gather((b + 1) % 3)

        r = rows[b]
        w_ref = ew_v[b]

        @plsc.parallel_loop(0, SB, 1, unroll=8)
        def _(i):
            w = plsc.load_gather(w_ref, [jnp.full((LANES,), i, jnp.int32)])
            for j in range(F // LANES):
                r[i, pl.ds(j * LANES, LANES)] = (
                    r[i, pl.ds(j * LANES, LANES)] * w)

        @pl.when(k - 1 >= lo)
        def _():
            drain_scatter((b + 2) % 3)

        @pl.when(k + 2 < hi)
        def _():
            fire_idx(k + 2, (b + 2) % 3)

        fire_scatter(b)

    fire_idx(lo, 0)
    zvec = jnp.zeros((LANES,), jnp.float32)
    zoff = 0
    for rbuf in rows:
        cnt = min(SB, ROWS_PER_TILE - zoff)
        if cnt <= 0:
            break

        @plsc.parallel_loop(0, cnt, 1, unroll=8)
        def _(i, rbuf=rbuf):
            for j in range(F // LANES):
                rbuf[i, pl.ds(j * LANES, LANES)] = zvec

        pltpu.sync_copy(rbuf.at[pl.ds(0, cnt)], acc.at[pl.ds(r0 + zoff, cnt)])
        zoff += cnt
    assert zoff == ROWS_PER_TILE
    wait_idx(0)
    fire_gather(0)
    fire_idx(lo + 1, 1)
    plsc.subcore_barrier()

    n_t = (hi - lo + 2) // 3

    def t_body(t, carry):
        k0 = lo + 3 * t
        phase(k0, 0)
        for b in (1, 2):
            @pl.when(k0 + b < hi)
            def _(b=b):
                phase(k0 + b, b)
        return carry

    lax.fori_loop(0, n_t, t_body, 0)

    for b in range(3):
        @pl.when((hi - 1 - lo) % 3 == b)
        def _(b=b):
            drain_scatter(b)

    plsc.subcore_barrier()
    pltpu.sync_copy(acc.at[pl.ds(r0, ROWS_PER_TILE)],
                    out_hbm.at[cid].at[pl.ds(r0, ROWS_PER_TILE)])


def _make_seg(F, CHUNK, NCH):
    SB = CHUNK * NCH
    return pl.kernel(
        functools.partial(_sc_seg_body, F, CHUNK, NCH),
        out_type=jax.ShapeDtypeStruct((NC, N_PAD, F), jnp.float32),
        mesh=plsc.VectorSubcoreMesh(core_axis_name="c", subcore_axis_name="s",
                                    num_cores=NC, num_subcores=NS),
        scratch_types=(
            [pltpu.VMEM_SHARED((N_PAD, F), jnp.float32)]
            + [pltpu.VMEM((NCH, CHUNK), jnp.int32)] * 3
            + [pltpu.VMEM((NCH, CHUNK), jnp.int32)] * 3
            + [pltpu.VMEM((SB,), jnp.float32)] * 3
            + [pltpu.VMEM((SB, F), jnp.float32)] * 3
            + [pltpu.SemaphoreType.DMA] * 9
        ),
        compiler_params=pltpu.CompilerParams(needs_layout_passes=False,
                                             use_tc_tiling_on_sc=False,
                                             skip_device_barrier=True),
    )


CHUNK16, NCH16 = 125, 8
CHUNK64, NCH64 = 128, 2
_seg16 = _make_seg(HIDDEN1, CHUNK16, NCH16)
_seg64 = _make_seg(OUT_DIM, CHUNK64, NCH64)


def _mm1_body(x_ref, w_ref, o_ref):
    o_ref[...] = jnp.dot(x_ref[...], w_ref[...],
                         preferred_element_type=jnp.float32)


_mm1 = pl.pallas_call(
    _mm1_body,
    out_shape=jax.ShapeDtypeStruct((N_NODES, HIDDEN1), jnp.float32),
)


def _mid_body(p_ref, w_ref, o_ref):
    h1 = jnp.maximum(p_ref[0, :N_NODES] + p_ref[1, :N_NODES], 0.0)
    o_ref[...] = jnp.dot(h1, w_ref[...], preferred_element_type=jnp.float32)


_mid = pl.pallas_call(
    _mid_body,
    out_shape=jax.ShapeDtypeStruct((N_NODES, OUT_DIM), jnp.float32),
)


def _soft_body(p_ref, o_ref):
    h = p_ref[0, :N_NODES] + p_ref[1, :N_NODES]
    m = jnp.max(h, axis=1, keepdims=True)
    e = jnp.exp(h - m)
    o_ref[...] = e / jnp.sum(e, axis=1, keepdims=True)


_soft = pl.pallas_call(
    _soft_body,
    out_shape=jax.ShapeDtypeStruct((N_NODES, OUT_DIM), jnp.float32),
)


@jax.jit
def _impl(x, src, dst, ew, W1, W2):
    src16 = src.reshape(N_EDGES // CHUNK16, CHUNK16)
    dst16 = dst.reshape(N_EDGES // CHUNK16, CHUNK16)
    src64 = src.reshape(N_EDGES // CHUNK64, CHUNK64)
    dst64 = dst.reshape(N_EDGES // CHUNK64, CHUNK64)
    hw1 = _mm1(x, W1)
    p1 = _seg16(hw1, src16, dst16, ew)
    hw2 = _mid(p1, W2)
    p2 = _seg64(hw2, src64, dst64, ew)
    return _soft(p2)


def kernel(x, edge_index, edge_weight, W1, W2):
    src = edge_index[0].astype(jnp.int32)
    dst = edge_index[1].astype(jnp.int32)
    return _impl(x, src, dst, edge_weight, W1, W2)


# speedup vs baseline: 19.1566x; 1.0131x over previous
import functools

import jax
import jax.numpy as jnp
from jax import lax
from jax.experimental import pallas as pl
from jax.experimental.pallas import tpu as pltpu
from jax.experimental.pallas import tpu_sc as plsc

N_NODES = 10000
N_EDGES = 320000
D_FEAT = 128
HIDDEN1 = 16
OUT_DIM = 64

LANES = 16
NC = 2
NS = 16
NW = NC * NS
ROWS_PER_TILE = 632
N_PAD = NS * ROWS_PER_TILE


def _sc_seg_body(F, CHUNK, NCH, h_hbm, src_hbm, dst_hbm, ew_hbm,
                 out_hbm, acc, src_v0, src_v1, src_v2, dst_v0, dst_v1, dst_v2,
                 ew_v0, ew_v1, ew_v2, rows0, rows1, rows2, sem_i0, sem_i1,
                 sem_i2, sem_g0, sem_g1, sem_g2, sem_s0, sem_s1, sem_s2):
    SB = CHUNK * NCH
    NSB = N_EDGES // SB
    cid = lax.axis_index("c")
    sid = lax.axis_index("s")
    wid = sid * NC + cid

    src_v = (src_v0, src_v1, src_v2)
    dst_v = (dst_v0, dst_v1, dst_v2)
    ew_v = (ew_v0, ew_v1, ew_v2)
    rows = (rows0, rows1, rows2)
    sem_i = (sem_i0, sem_i1, sem_i2)
    sem_g = (sem_g0, sem_g1, sem_g2)
    sem_s = (sem_s0, sem_s1, sem_s2)

    r0 = sid * ROWS_PER_TILE
    lo = (wid * NSB) // NW
    hi = ((wid + 1) * NSB) // NW

    def fire_idx(k, b):
        pltpu.async_copy(src_hbm.at[pl.ds(k * NCH, NCH)], src_v[b], sem_i[b])
        pltpu.async_copy(dst_hbm.at[pl.ds(k * NCH, NCH)], dst_v[b], sem_i[b])
        pltpu.async_copy(ew_hbm.at[pl.ds(k * SB, SB)], ew_v[b], sem_i[b])

    def wait_idx(b):
        pltpu.make_async_copy(src_hbm.at[pl.ds(0, NCH)], src_v[b],
                              sem_i[b]).wait()
        pltpu.make_async_copy(dst_hbm.at[pl.ds(0, NCH)], dst_v[b],
                              sem_i[b]).wait()
        pltpu.make_async_copy(ew_hbm.at[pl.ds(0, SB)], ew_v[b],
                              sem_i[b]).wait()

    def fire_gather(b):
        for j in range(NCH):
            pltpu.async_copy(h_hbm.at[src_v[b].at[j]],
                             rows[b].at[pl.ds(j * CHUNK, CHUNK)], sem_g[b])

    def wait_gather(b):
        for j in range(NCH):
            pltpu.make_async_copy(h_hbm.at[src_v[b].at[j]],
                                  rows[b].at[pl.ds(j * CHUNK, CHUNK)],
                                  sem_g[b]).wait()

    def fire_scatter(b):
        for j in range(NCH):
            pltpu.async_copy(rows[b].at[pl.ds(j * CHUNK, CHUNK)],
                             acc.at[dst_v[b].at[j]], sem_s[b], add=True)

    def drain_scatter(b):
        for j in range(NCH):
            pltpu.make_async_copy(rows[b].at[pl.ds(j * CHUNK, CHUNK)],
                                  acc.at[dst_v[b].at[j]], sem_s[b]).wait()

    def phase(k, b):
        wait_gather(b)

        @pl.when(k + 1 < hi)
        def _():
            wait_idx((b + 1) % 3)
            fire_gather((b + 1) % 3)

        r = rows[b]
        w_ref = ew_v[b]

        @plsc.parallel_loop(0, SB, 1, unroll=8)
        def _(i):
            w = plsc.load_gather(w_ref, [jnp.full((LANES,), i, jnp.int32)])
            for j in range(F // LANES):
                r[i, pl.ds(j * LANES, LANES)] = (
                    r[i, pl.ds(j * LANES, LANES)] * w)

        @pl.when(k - 1 >= lo)
        def _():
            drain_scatter((b + 2) % 3)

        @pl.when(k + 2 < hi)
        def _():
            fire_idx(k + 2, (b + 2) % 3)

        fire_scatter(b)

    fire_idx(lo, 0)
    zvec = jnp.zeros((LANES,), jnp.float32)
    zoff = 0
    for rbuf in rows:
        cnt = min(SB, ROWS_PER_TILE - zoff)
        if cnt <= 0:
            break

        @plsc.parallel_loop(0, cnt, 1, unroll=8)
        def _(i, rbuf=rbuf):
            for j in range(F // LANES):
                rbuf[i, pl.ds(j * LANES, LANES)] = zvec

        pltpu.sync_copy(rbuf.at[pl.ds(0, cnt)], acc.at[pl.ds(r0 + zoff, cnt)])
        zoff += cnt
    assert zoff == ROWS_PER_TILE
    wait_idx(0)
    fire_gather(0)
    fire_idx(lo + 1, 1)
    plsc.subcore_barrier()

    n_t = (hi - lo + 2) // 3

    def t_body(t, carry):
        k0 = lo + 3 * t
        phase(k0, 0)
        for b in (1, 2):
            @pl.when(k0 + b < hi)
            def _(b=b):
                phase(k0 + b, b)
        return carry

    lax.fori_loop(0, n_t, t_body, 0)

    for b in range(3):
        @pl.when((hi - 1 - lo) % 3 == b)
        def _(b=b):
            drain_scatter(b)

    plsc.subcore_barrier()
    pltpu.sync_copy(acc.at[pl.ds(r0, ROWS_PER_TILE)],
                    out_hbm.at[cid].at[pl.ds(r0, ROWS_PER_TILE)])


def _make_seg(F, CHUNK, NCH):
    SB = CHUNK * NCH
    return pl.kernel(
        functools.partial(_sc_seg_body, F, CHUNK, NCH),
        out_type=jax.ShapeDtypeStruct((NC, N_PAD, F), jnp.float32),
        mesh=plsc.VectorSubcoreMesh(core_axis_name="c", subcore_axis_name="s",
                                    num_cores=NC, num_subcores=NS),
        scratch_types=(
            [pltpu.VMEM_SHARED((N_PAD, F), jnp.float32)]
            + [pltpu.VMEM((NCH, CHUNK), jnp.int32)] * 3
            + [pltpu.VMEM((NCH, CHUNK), jnp.int32)] * 3
            + [pltpu.VMEM((SB,), jnp.float32)] * 3
            + [pltpu.VMEM((SB, F), jnp.float32)] * 3
            + [pltpu.SemaphoreType.DMA] * 9
        ),
        compiler_params=pltpu.CompilerParams(needs_layout_passes=False,
                                             use_tc_tiling_on_sc=False,
                                             skip_device_barrier=True),
    )


CHUNK16, NCH16 = 128, 5
CHUNK64, NCH64 = 128, 2
_seg16 = _make_seg(HIDDEN1, CHUNK16, NCH16)
_seg64 = _make_seg(OUT_DIM, CHUNK64, NCH64)


def _mm1_body(x_ref, w_ref, o_ref):
    o_ref[...] = jnp.dot(x_ref[...], w_ref[...],
                         preferred_element_type=jnp.float32)


_mm1 = pl.pallas_call(
    _mm1_body,
    out_shape=jax.ShapeDtypeStruct((N_NODES, HIDDEN1), jnp.float32),
)


def _mid_body(p_ref, w_ref, o_ref):
    h1 = jnp.maximum(p_ref[0, :N_NODES] + p_ref[1, :N_NODES], 0.0)
    o_ref[...] = jnp.dot(h1, w_ref[...], preferred_element_type=jnp.float32)


_mid = pl.pallas_call(
    _mid_body,
    out_shape=jax.ShapeDtypeStruct((N_NODES, OUT_DIM), jnp.float32),
)


def _soft_body(p_ref, o_ref):
    h = p_ref[0, :N_NODES] + p_ref[1, :N_NODES]
    m = jnp.max(h, axis=1, keepdims=True)
    e = jnp.exp(h - m)
    o_ref[...] = e / jnp.sum(e, axis=1, keepdims=True)


_soft = pl.pallas_call(
    _soft_body,
    out_shape=jax.ShapeDtypeStruct((N_NODES, OUT_DIM), jnp.float32),
)


@jax.jit
def _impl(x, src, dst, ew, W1, W2):
    src2 = src.reshape(N_EDGES // 128, 128)
    dst2 = dst.reshape(N_EDGES // 128, 128)
    hw1 = _mm1(x, W1)
    p1 = _seg16(hw1, src2, dst2, ew)
    hw2 = _mid(p1, W2)
    p2 = _seg64(hw2, src2, dst2, ew)
    return _soft(p2)


def kernel(x, edge_index, edge_weight, W1, W2):
    src = edge_index[0].astype(jnp.int32)
    dst = edge_index[1].astype(jnp.int32)
    return _impl(x, src, dst, edge_weight, W1, W2)


# speedup vs baseline: 20.7328x; 1.0823x over previous
import functools

import jax
import jax.numpy as jnp
from jax import lax
from jax.experimental import pallas as pl
from jax.experimental.pallas import tpu as pltpu
from jax.experimental.pallas import tpu_sc as plsc

N_NODES = 10000
N_EDGES = 320000
D_FEAT = 128
HIDDEN1 = 16
OUT_DIM = 64

LANES = 16
NC = 2
NS = 16
NW = NC * NS
ROWS_PER_TILE = 632
N_PAD = NS * ROWS_PER_TILE


def _sc_seg_body(F, CHUNK, NCH, h_hbm, ei_hbm, ew_hbm,
                 out_hbm, acc, src_v0, src_v1, src_v2, dst_v0, dst_v1, dst_v2,
                 ew_v0, ew_v1, ew_v2, rows0, rows1, rows2, sem_i0, sem_i1,
                 sem_i2, sem_g0, sem_g1, sem_g2, sem_s0, sem_s1, sem_s2):
    SB = CHUNK * NCH
    NSB = N_EDGES // SB
    cid = lax.axis_index("c")
    sid = lax.axis_index("s")
    wid = sid * NC + cid

    src_v = (src_v0, src_v1, src_v2)
    dst_v = (dst_v0, dst_v1, dst_v2)
    ew_v = (ew_v0, ew_v1, ew_v2)
    rows = (rows0, rows1, rows2)
    sem_i = (sem_i0, sem_i1, sem_i2)
    sem_g = (sem_g0, sem_g1, sem_g2)
    sem_s = (sem_s0, sem_s1, sem_s2)

    r0 = sid * ROWS_PER_TILE
    lo = (wid * NSB) // NW
    hi = ((wid + 1) * NSB) // NW

    def fire_idx(k, b):
        pltpu.async_copy(ei_hbm.at[0, pl.ds(k * NCH, NCH)], src_v[b],
                         sem_i[b])
        pltpu.async_copy(ei_hbm.at[1, pl.ds(k * NCH, NCH)], dst_v[b],
                         sem_i[b])
        pltpu.async_copy(ew_hbm.at[pl.ds(k * SB, SB)], ew_v[b], sem_i[b])

    def wait_idx(b):
        pltpu.make_async_copy(ei_hbm.at[0, pl.ds(0, NCH)], src_v[b],
                              sem_i[b]).wait()
        pltpu.make_async_copy(ei_hbm.at[1, pl.ds(0, NCH)], dst_v[b],
                              sem_i[b]).wait()
        pltpu.make_async_copy(ew_hbm.at[pl.ds(0, SB)], ew_v[b],
                              sem_i[b]).wait()

    def fire_gather(b):
        for j in range(NCH):
            pltpu.async_copy(h_hbm.at[src_v[b].at[j]],
                             rows[b].at[pl.ds(j * CHUNK, CHUNK)], sem_g[b])

    def wait_gather(b):
        for j in range(NCH):
            pltpu.make_async_copy(h_hbm.at[src_v[b].at[j]],
                                  rows[b].at[pl.ds(j * CHUNK, CHUNK)],
                                  sem_g[b]).wait()

    def fire_scatter(b):
        for j in range(NCH):
            pltpu.async_copy(rows[b].at[pl.ds(j * CHUNK, CHUNK)],
                             acc.at[dst_v[b].at[j]], sem_s[b], add=True)

    def drain_scatter(b):
        for j in range(NCH):
            pltpu.make_async_copy(rows[b].at[pl.ds(j * CHUNK, CHUNK)],
                                  acc.at[dst_v[b].at[j]], sem_s[b]).wait()

    def phase(k, b):
        wait_gather(b)

        @pl.when(k + 1 < hi)
        def _():
            wait_idx((b + 1) % 3)
            fire_gather((b + 1) % 3)

        r = rows[b]
        w_ref = ew_v[b]

        @plsc.parallel_loop(0, SB, 1, unroll=8)
        def _(i):
            w = plsc.load_gather(w_ref, [jnp.full((LANES,), i, jnp.int32)])
            for j in range(F // LANES):
                r[i, pl.ds(j * LANES, LANES)] = (
                    r[i, pl.ds(j * LANES, LANES)] * w)

        @pl.when(k - 1 >= lo)
        def _():
            drain_scatter((b + 2) % 3)

        @pl.when(k + 2 < hi)
        def _():
            fire_idx(k + 2, (b + 2) % 3)

        fire_scatter(b)

    fire_idx(lo, 0)
    zvec = jnp.zeros((LANES,), jnp.float32)
    zoff = 0
    for rbuf in rows:
        cnt = min(SB, ROWS_PER_TILE - zoff)
        if cnt <= 0:
            break

        @plsc.parallel_loop(0, cnt, 1, unroll=8)
        def _(i, rbuf=rbuf):
            for j in range(F // LANES):
                rbuf[i, pl.ds(j * LANES, LANES)] = zvec

        pltpu.sync_copy(rbuf.at[pl.ds(0, cnt)], acc.at[pl.ds(r0 + zoff, cnt)])
        zoff += cnt
    assert zoff == ROWS_PER_TILE
    wait_idx(0)
    fire_gather(0)
    fire_idx(lo + 1, 1)
    plsc.subcore_barrier()

    n_t = (hi - lo + 2) // 3

    def t_body(t, carry):
        k0 = lo + 3 * t
        phase(k0, 0)
        for b in (1, 2):
            @pl.when(k0 + b < hi)
            def _(b=b):
                phase(k0 + b, b)
        return carry

    lax.fori_loop(0, n_t, t_body, 0)

    for b in range(3):
        @pl.when((hi - 1 - lo) % 3 == b)
        def _(b=b):
            drain_scatter(b)

    plsc.subcore_barrier()
    pltpu.sync_copy(acc.at[pl.ds(r0, ROWS_PER_TILE)],
                    out_hbm.at[cid].at[pl.ds(r0, ROWS_PER_TILE)])


def _make_seg(F, CHUNK, NCH):
    SB = CHUNK * NCH
    return pl.kernel(
        functools.partial(_sc_seg_body, F, CHUNK, NCH),
        out_type=jax.ShapeDtypeStruct((NC, N_PAD, F), jnp.float32),
        mesh=plsc.VectorSubcoreMesh(core_axis_name="c", subcore_axis_name="s",
                                    num_cores=NC, num_subcores=NS),
        scratch_types=(
            [pltpu.VMEM_SHARED((N_PAD, F), jnp.float32)]
            + [pltpu.VMEM((NCH, CHUNK), jnp.int32)] * 3
            + [pltpu.VMEM((NCH, CHUNK), jnp.int32)] * 3
            + [pltpu.VMEM((SB,), jnp.float32)] * 3
            + [pltpu.VMEM((SB, F), jnp.float32)] * 3
            + [pltpu.SemaphoreType.DMA] * 9
        ),
        compiler_params=pltpu.CompilerParams(needs_layout_passes=False,
                                             use_tc_tiling_on_sc=False,
                                             skip_device_barrier=True),
    )


CHUNK16, NCH16 = 128, 5
CHUNK64, NCH64 = 128, 2
_seg16 = _make_seg(HIDDEN1, CHUNK16, NCH16)
_seg64 = _make_seg(OUT_DIM, CHUNK64, NCH64)


def _mm1_body(x_ref, w_ref, o_ref):
    w = w_ref[...]
    for s in range(8):
        hs = jnp.dot(x_ref[:, s, :], w, preferred_element_type=jnp.float32)
        o_ref[:, s * HIDDEN1:(s + 1) * HIDDEN1] = hs


_mm1 = pl.pallas_call(
    _mm1_body,
    out_shape=jax.ShapeDtypeStruct((N_NODES * HIDDEN1 // 128, 128),
                                   jnp.float32),
)


def _mid_body(p_ref, w_ref, o_ref):
    h1 = jnp.maximum(p_ref[0, :N_NODES] + p_ref[1, :N_NODES], 0.0)
    o_ref[...] = jnp.dot(h1, w_ref[...], preferred_element_type=jnp.float32)


_mid = pl.pallas_call(
    _mid_body,
    out_shape=jax.ShapeDtypeStruct((N_NODES, OUT_DIM), jnp.float32),
)


def _soft_body(p_ref, o_ref):
    h = p_ref[0, :N_NODES] + p_ref[1, :N_NODES]
    m = jnp.max(h, axis=1, keepdims=True)
    e = jnp.exp(h - m)
    o_ref[...] = e / jnp.sum(e, axis=1, keepdims=True)


_soft = pl.pallas_call(
    _soft_body,
    out_shape=jax.ShapeDtypeStruct((N_NODES, OUT_DIM), jnp.float32),
)


@jax.jit
def _impl(x, edge_index, ew, W1, W2):
    ei = edge_index.astype(jnp.int32).reshape(2, N_EDGES // 128, 128)
    x3 = x.reshape(N_NODES // 8, 8, D_FEAT)
    hw1 = _mm1(x3, W1).reshape(N_NODES, HIDDEN1)
    p1 = _seg16(hw1, ei, ew)
    hw2 = _mid(p1, W2)
    p2 = _seg64(hw2, ei, ew)
    return _soft(p2)


def kernel(x, edge_index, edge_weight, W1, W2):
    return _impl(x, edge_index, edge_weight, W1, W2)


# speedup vs baseline: 23.1187x; 1.1151x over previous
import functools

import jax
import jax.numpy as jnp
from jax import lax
from jax.experimental import pallas as pl
from jax.experimental.pallas import tpu as pltpu
from jax.experimental.pallas import tpu_sc as plsc

N_NODES = 10000
N_EDGES = 320000
D_FEAT = 128
HIDDEN1 = 16
OUT_DIM = 64

LANES = 16
NC = 2
NS = 16
NW = NC * NS
ROWS_PER_TILE = 632
N_PAD = NS * ROWS_PER_TILE


def _sc_seg_body(F, CHUNK, NCH, PERMUTED, h_hbm, ei_hbm, ew_hbm,
                 out_hbm, acc, src_v0, src_v1, src_v2, dst_v0, dst_v1, dst_v2,
                 ew_v0, ew_v1, ew_v2, rows0, rows1, rows2, sem_i0, sem_i1,
                 sem_i2, sem_g0, sem_g1, sem_g2, sem_s0, sem_s1, sem_s2):
    SB = CHUNK * NCH
    NSB = N_EDGES // SB
    cid = lax.axis_index("c")
    sid = lax.axis_index("s")
    wid = sid * NC + cid

    src_v = (src_v0, src_v1, src_v2)
    dst_v = (dst_v0, dst_v1, dst_v2)
    ew_v = (ew_v0, ew_v1, ew_v2)
    rows = (rows0, rows1, rows2)
    sem_i = (sem_i0, sem_i1, sem_i2)
    sem_g = (sem_g0, sem_g1, sem_g2)
    sem_s = (sem_s0, sem_s1, sem_s2)

    r0 = sid * ROWS_PER_TILE
    lo = (wid * NSB) // NW
    hi = ((wid + 1) * NSB) // NW

    def fire_idx(k, b):
        pltpu.async_copy(ei_hbm.at[0, pl.ds(k * NCH, NCH)], src_v[b],
                         sem_i[b])
        pltpu.async_copy(ei_hbm.at[1, pl.ds(k * NCH, NCH)], dst_v[b],
                         sem_i[b])
        pltpu.async_copy(ew_hbm.at[pl.ds(k * SB, SB)], ew_v[b], sem_i[b])

    def wait_idx(b):
        pltpu.make_async_copy(ei_hbm.at[0, pl.ds(0, NCH)], src_v[b],
                              sem_i[b]).wait()
        pltpu.make_async_copy(ei_hbm.at[1, pl.ds(0, NCH)], dst_v[b],
                              sem_i[b]).wait()
        pltpu.make_async_copy(ew_hbm.at[pl.ds(0, SB)], ew_v[b],
                              sem_i[b]).wait()

    def fire_gather(b):
        if PERMUTED:
            for j in range(NCH):
                for v in range(CHUNK // LANES):
                    r = src_v[b][j, pl.ds(v * LANES, LANES)]
                    pos = (((r >> 1) & 3) * 2528
                           + ((r >> 3) << 1) + (r & 1))
                    src_v[b][j, pl.ds(v * LANES, LANES)] = pos
        for j in range(NCH):
            pltpu.async_copy(h_hbm.at[src_v[b].at[j]],
                             rows[b].at[pl.ds(j * CHUNK, CHUNK)], sem_g[b])

    def wait_gather(b):
        for j in range(NCH):
            pltpu.make_async_copy(h_hbm.at[src_v[b].at[j]],
                                  rows[b].at[pl.ds(j * CHUNK, CHUNK)],
                                  sem_g[b]).wait()

    def fire_scatter(b):
        for j in range(NCH):
            pltpu.async_copy(rows[b].at[pl.ds(j * CHUNK, CHUNK)],
                             acc.at[dst_v[b].at[j]], sem_s[b], add=True)

    def drain_scatter(b):
        for j in range(NCH):
            pltpu.make_async_copy(rows[b].at[pl.ds(j * CHUNK, CHUNK)],
                                  acc.at[dst_v[b].at[j]], sem_s[b]).wait()

    def phase(k, b):
        wait_gather(b)

        @pl.when(k + 1 < hi)
        def _():
            wait_idx((b + 1) % 3)
            fire_gather((b + 1) % 3)

        r = rows[b]
        w_ref = ew_v[b]

        @plsc.parallel_loop(0, SB, 1, unroll=8)
        def _(i):
            w = plsc.load_gather(w_ref, [jnp.full((LANES,), i, jnp.int32)])
            for j in range(F // LANES):
                r[i, pl.ds(j * LANES, LANES)] = (
                    r[i, pl.ds(j * LANES, LANES)] * w)

        @pl.when(k - 1 >= lo)
        def _():
            drain_scatter((b + 2) % 3)

        @pl.when(k + 2 < hi)
        def _():
            fire_idx(k + 2, (b + 2) % 3)

        fire_scatter(b)

    fire_idx(lo, 0)
    zvec = jnp.zeros((LANES,), jnp.float32)
    zoff = 0
    for rbuf in rows:
        cnt = min(SB, ROWS_PER_TILE - zoff)
        if cnt <= 0:
            break

        @plsc.parallel_loop(0, cnt, 1, unroll=8)
        def _(i, rbuf=rbuf):
            for j in range(F // LANES):
                rbuf[i, pl.ds(j * LANES, LANES)] = zvec

        pltpu.sync_copy(rbuf.at[pl.ds(0, cnt)], acc.at[pl.ds(r0 + zoff, cnt)])
        zoff += cnt
    assert zoff == ROWS_PER_TILE
    wait_idx(0)
    fire_gather(0)
    fire_idx(lo + 1, 1)
    plsc.subcore_barrier()

    n_t = (hi - lo + 2) // 3

    def t_body(t, carry):
        k0 = lo + 3 * t
        phase(k0, 0)
        for b in (1, 2):
            @pl.when(k0 + b < hi)
            def _(b=b):
                phase(k0 + b, b)
        return carry

    lax.fori_loop(0, n_t, t_body, 0)

    for b in range(3):
        @pl.when((hi - 1 - lo) % 3 == b)
        def _(b=b):
            drain_scatter(b)

    plsc.subcore_barrier()
    pltpu.sync_copy(acc.at[pl.ds(r0, ROWS_PER_TILE)],
                    out_hbm.at[cid].at[pl.ds(r0, ROWS_PER_TILE)])


def _make_seg(F, CHUNK, NCH, PERMUTED=False):
    SB = CHUNK * NCH
    return pl.kernel(
        functools.partial(_sc_seg_body, F, CHUNK, NCH, PERMUTED),
        out_type=jax.ShapeDtypeStruct((NC, N_PAD, F), jnp.float32),
        mesh=plsc.VectorSubcoreMesh(core_axis_name="c", subcore_axis_name="s",
                                    num_cores=NC, num_subcores=NS),
        scratch_types=(
            [pltpu.VMEM_SHARED((N_PAD, F), jnp.float32)]
            + [pltpu.VMEM((NCH, CHUNK), jnp.int32)] * 3
            + [pltpu.VMEM((NCH, CHUNK), jnp.int32)] * 3
            + [pltpu.VMEM((SB,), jnp.float32)] * 3
            + [pltpu.VMEM((SB, F), jnp.float32)] * 3
            + [pltpu.SemaphoreType.DMA] * 9
        ),
        compiler_params=pltpu.CompilerParams(needs_layout_passes=False,
                                             use_tc_tiling_on_sc=False,
                                             skip_device_barrier=True),
    )


CHUNK16, NCH16 = 128, 5
CHUNK64, NCH64 = 128, 2
_seg16 = _make_seg(HIDDEN1, CHUNK16, NCH16)
_seg64 = _make_seg(OUT_DIM, CHUNK64, NCH64, PERMUTED=True)


def _mm1_body(x_ref, w_ref, o_ref):
    w = w_ref[...]
    for s in range(8):
        hs = jnp.dot(x_ref[:, s, :], w, preferred_element_type=jnp.float32)
        o_ref[:, s * HIDDEN1:(s + 1) * HIDDEN1] = hs


_mm1 = pl.pallas_call(
    _mm1_body,
    out_shape=jax.ShapeDtypeStruct((N_NODES * HIDDEN1 // 128, 128),
                                   jnp.float32),
)


def _mid_body(p_ref, w_ref, o_ref):
    h = jnp.maximum(p_ref[0] + p_ref[1], 0.0)
    w = w_ref[...]
    for t in range(4):
        ha = jnp.dot(h[:, (2 * t) * HIDDEN1:(2 * t + 1) * HIDDEN1], w,
                     preferred_element_type=jnp.float32)
        hb = jnp.dot(h[:, (2 * t + 1) * HIDDEN1:(2 * t + 2) * HIDDEN1], w,
                     preferred_element_type=jnp.float32)
        o_ref[pl.ds(t * (N_PAD // 8), N_PAD // 8), :] = (
            jnp.concatenate([ha, hb], axis=1))


_mid = pl.pallas_call(
    _mid_body,
    out_shape=jax.ShapeDtypeStruct((N_PAD // 2, 2 * OUT_DIM), jnp.float32),
)


def _soft_body(p_ref, o_ref):
    h = p_ref[0, :N_NODES // 2] + p_ref[1, :N_NODES // 2]
    outs = []
    for u in range(2):
        hu = h[:, u * OUT_DIM:(u + 1) * OUT_DIM]
        m = jnp.max(hu, axis=1, keepdims=True)
        e = jnp.exp(hu - m)
        outs.append(e / jnp.sum(e, axis=1, keepdims=True))
    o_ref[...] = jnp.stack(outs, axis=1).reshape(N_NODES, OUT_DIM)


_soft = pl.pallas_call(
    _soft_body,
    out_shape=jax.ShapeDtypeStruct((N_NODES, OUT_DIM), jnp.float32),
)


@jax.jit
def _impl(x, edge_index, ew, W1, W2):
    ei = edge_index.astype(jnp.int32).reshape(2, N_EDGES // 128, 128)
    x3 = x.reshape(N_NODES // 8, 8, D_FEAT)
    hw1 = _mm1(x3, W1).reshape(N_NODES, HIDDEN1)
    p1 = _seg16(hw1, ei, ew)
    hw2 = _mid(p1.reshape(NC, N_PAD // 8, 8 * HIDDEN1),
               W2).reshape(N_PAD, OUT_DIM)
    p2 = _seg64(hw2, ei, ew)
    return _soft(p2.reshape(NC, N_PAD // 2, 2 * OUT_DIM))


def kernel(x, edge_index, edge_weight, W1, W2):
    return _impl(x, edge_index, edge_weight, W1, W2)


# speedup vs baseline: 23.3119x; 1.0084x over previous
import functools

import jax
import jax.numpy as jnp
from jax import lax
from jax.experimental import pallas as pl
from jax.experimental.pallas import tpu as pltpu
from jax.experimental.pallas import tpu_sc as plsc

N_NODES = 10000
N_EDGES = 320000
D_FEAT = 128
HIDDEN1 = 16
OUT_DIM = 64

LANES = 16
NC = 2
NS = 16
NW = NC * NS
ROWS_PER_TILE = 632
N_PAD = NS * ROWS_PER_TILE


def _sc_seg_body(F, CHUNK, NCH, PERMUTED, h_hbm, ei_hbm, ew_hbm,
                 out_hbm, acc, src_v0, src_v1, src_v2, dst_v0, dst_v1, dst_v2,
                 ew_v0, ew_v1, ew_v2, rows0, rows1, rows2, sem_i0, sem_i1,
                 sem_i2, sem_g0, sem_g1, sem_g2, sem_s0, sem_s1, sem_s2):
    SB = CHUNK * NCH
    NSB = N_EDGES // SB
    cid = lax.axis_index("c")
    sid = lax.axis_index("s")
    wid = sid * NC + cid

    src_v = (src_v0, src_v1, src_v2)
    dst_v = (dst_v0, dst_v1, dst_v2)
    ew_v = (ew_v0, ew_v1, ew_v2)
    rows = (rows0, rows1, rows2)
    sem_i = (sem_i0, sem_i1, sem_i2)
    sem_g = (sem_g0, sem_g1, sem_g2)
    sem_s = (sem_s0, sem_s1, sem_s2)

    r0 = sid * ROWS_PER_TILE
    lo = (wid * NSB) // NW
    hi = ((wid + 1) * NSB) // NW

    def fire_idx(k, b):
        pltpu.async_copy(ei_hbm.at[0, pl.ds(k * NCH, NCH)], src_v[b],
                         sem_i[b])
        pltpu.async_copy(ei_hbm.at[1, pl.ds(k * NCH, NCH)], dst_v[b],
                         sem_i[b])
        pltpu.async_copy(ew_hbm.at[pl.ds(k * SB, SB)], ew_v[b], sem_i[b])

    def wait_idx(b):
        pltpu.make_async_copy(ei_hbm.at[0, pl.ds(0, NCH)], src_v[b],
                              sem_i[b]).wait()
        pltpu.make_async_copy(ei_hbm.at[1, pl.ds(0, NCH)], dst_v[b],
                              sem_i[b]).wait()
        pltpu.make_async_copy(ew_hbm.at[pl.ds(0, SB)], ew_v[b],
                              sem_i[b]).wait()

    def fire_gather(b):
        if PERMUTED:
            for j in range(NCH):
                for v in range(CHUNK // LANES):
                    r = src_v[b][j, pl.ds(v * LANES, LANES)]
                    pos = (((r >> 1) & 3) * 2528
                           + ((r >> 3) << 1) + (r & 1))
                    src_v[b][j, pl.ds(v * LANES, LANES)] = pos
        for j in range(NCH):
            pltpu.async_copy(h_hbm.at[src_v[b].at[j]],
                             rows[b].at[pl.ds(j * CHUNK, CHUNK)], sem_g[b])

    def wait_gather(b):
        for j in range(NCH):
            pltpu.make_async_copy(h_hbm.at[src_v[b].at[j]],
                                  rows[b].at[pl.ds(j * CHUNK, CHUNK)],
                                  sem_g[b]).wait()

    def fire_scatter(b):
        for j in range(NCH):
            pltpu.async_copy(rows[b].at[pl.ds(j * CHUNK, CHUNK)],
                             acc.at[dst_v[b].at[j]], sem_s[b], add=True)

    def drain_scatter(b):
        for j in range(NCH):
            pltpu.make_async_copy(rows[b].at[pl.ds(j * CHUNK, CHUNK)],
                                  acc.at[dst_v[b].at[j]], sem_s[b]).wait()

    def phase(k, b):
        wait_gather(b)

        @pl.when(k + 1 < hi)
        def _():
            wait_idx((b + 1) % 3)
            fire_gather((b + 1) % 3)

        r = rows[b]
        w_ref = ew_v[b]

        @plsc.parallel_loop(0, SB, 1, unroll=8)
        def _(i):
            w = plsc.load_gather(w_ref, [jnp.full((LANES,), i, jnp.int32)])
            for j in range(F // LANES):
                r[i, pl.ds(j * LANES, LANES)] = (
                    r[i, pl.ds(j * LANES, LANES)] * w)

        @pl.when(k - 1 >= lo)
        def _():
            drain_scatter((b + 2) % 3)

        @pl.when(k + 2 < hi)
        def _():
            fire_idx(k + 2, (b + 2) % 3)

        fire_scatter(b)

    fire_idx(lo, 0)
    zvec = jnp.zeros((LANES,), jnp.float32)
    zoff = 0
    for rbuf in rows:
        cnt = min(SB, ROWS_PER_TILE - zoff)
        if cnt <= 0:
            break

        @plsc.parallel_loop(0, cnt, 1, unroll=8)
        def _(i, rbuf=rbuf):
            for j in range(F // LANES):
                rbuf[i, pl.ds(j * LANES, LANES)] = zvec

        pltpu.sync_copy(rbuf.at[pl.ds(0, cnt)], acc.at[pl.ds(r0 + zoff, cnt)])
        zoff += cnt
    assert zoff == ROWS_PER_TILE
    wait_idx(0)
    fire_gather(0)
    fire_idx(lo + 1, 1)
    plsc.subcore_barrier()

    n_t = (hi - lo + 2) // 3

    def t_body(t, carry):
        k0 = lo + 3 * t
        phase(k0, 0)
        for b in (1, 2):
            @pl.when(k0 + b < hi)
            def _(b=b):
                phase(k0 + b, b)
        return carry

    lax.fori_loop(0, n_t, t_body, 0)

    for b in range(3):
        @pl.when((hi - 1 - lo) % 3 == b)
        def _(b=b):
            drain_scatter(b)

    plsc.subcore_barrier()
    pltpu.sync_copy(acc.at[pl.ds(r0, ROWS_PER_TILE)],
                    out_hbm.at[cid].at[pl.ds(r0, ROWS_PER_TILE)])


def _make_seg(F, CHUNK, NCH, PERMUTED=False):
    SB = CHUNK * NCH
    return pl.kernel(
        functools.partial(_sc_seg_body, F, CHUNK, NCH, PERMUTED),
        out_type=jax.ShapeDtypeStruct((NC, N_PAD, F), jnp.float32),
        mesh=plsc.VectorSubcoreMesh(core_axis_name="c", subcore_axis_name="s",
                                    num_cores=NC, num_subcores=NS),
        scratch_types=(
            [pltpu.VMEM_SHARED((N_PAD, F), jnp.float32)]
            + [pltpu.VMEM((NCH, CHUNK), jnp.int32)] * 3
            + [pltpu.VMEM((NCH, CHUNK), jnp.int32)] * 3
            + [pltpu.VMEM((SB,), jnp.float32)] * 3
            + [pltpu.VMEM((SB, F), jnp.float32)] * 3
            + [pltpu.SemaphoreType.DMA] * 9
        ),
        compiler_params=pltpu.CompilerParams(needs_layout_passes=False,
                                             use_tc_tiling_on_sc=False,
                                             skip_device_barrier=True),
    )


CHUNK16, NCH16 = 128, 5
CHUNK64, NCH64 = 128, 2
_seg16 = _make_seg(HIDDEN1, CHUNK16, NCH16)
_seg64 = _make_seg(OUT_DIM, CHUNK64, NCH64, PERMUTED=True)


def _mm1_body(x_ref, w_ref, o_ref):
    w = w_ref[...]
    for s in range(8):
        hs = jnp.dot(x_ref[:, s, :], w, preferred_element_type=jnp.float32)
        o_ref[:, s * HIDDEN1:(s + 1) * HIDDEN1] = hs


_mm1 = pl.pallas_call(
    _mm1_body,
    out_shape=jax.ShapeDtypeStruct((N_NODES * HIDDEN1 // 128, 128),
                                   jnp.float32),
)


def _mid_body(p_ref, w_ref, o_ref):
    h = jnp.maximum(p_ref[0] + p_ref[1], 0.0)
    w = w_ref[...]
    for t in range(4):
        ha = jnp.dot(h[:, (2 * t) * HIDDEN1:(2 * t + 1) * HIDDEN1], w,
                     preferred_element_type=jnp.float32)
        hb = jnp.dot(h[:, (2 * t + 1) * HIDDEN1:(2 * t + 2) * HIDDEN1], w,
                     preferred_element_type=jnp.float32)
        o_ref[pl.ds(t * (N_PAD // 8), N_PAD // 8), :] = (
            jnp.concatenate([ha, hb], axis=1))


_mid = pl.pallas_call(
    _mid_body,
    out_shape=jax.ShapeDtypeStruct((N_PAD // 2, 2 * OUT_DIM), jnp.float32),
)


_SOFT_GRID = 4
_SOFT_B = N_PAD // 2 // _SOFT_GRID


def _soft_body(p_ref, o_ref):
    h = p_ref[0] + p_ref[1]
    outs = []
    for u in range(2):
        hu = h[:, u * OUT_DIM:(u + 1) * OUT_DIM]
        m = jnp.max(hu, axis=1, keepdims=True)
        e = jnp.exp(hu - m)
        outs.append(e / jnp.sum(e, axis=1, keepdims=True))
    o_ref[...] = jnp.stack(outs, axis=1).reshape(2 * _SOFT_B, OUT_DIM)


_soft = pl.pallas_call(
    _soft_body,
    grid=(_SOFT_GRID,),
    in_specs=[pl.BlockSpec((2, _SOFT_B, 128), lambda i: (0, i, 0))],
    out_specs=pl.BlockSpec((2 * _SOFT_B, OUT_DIM), lambda i: (i, 0)),
    out_shape=jax.ShapeDtypeStruct((N_PAD, OUT_DIM), jnp.float32),
)


@jax.jit
def _impl(x, edge_index, ew, W1, W2):
    ei = edge_index.astype(jnp.int32).reshape(2, N_EDGES // 128, 128)
    x3 = x.reshape(N_NODES // 8, 8, D_FEAT)
    hw1 = _mm1(x3, W1).reshape(N_NODES, HIDDEN1)
    p1 = _seg16(hw1, ei, ew)
    hw2 = _mid(p1.reshape(NC, N_PAD // 8, 8 * HIDDEN1),
               W2).reshape(N_PAD, OUT_DIM)
    p2 = _seg64(hw2, ei, ew)
    return _soft(p2.reshape(NC, N_PAD // 2, 2 * OUT_DIM))[:N_NODES]


def kernel(x, edge_index, edge_weight, W1, W2):
    return _impl(x, edge_index, edge_weight, W1, W2)
